# baseline (speedup 1.0000x reference)
import functools

import jax
import jax.numpy as jnp
from jax import lax
from jax.experimental import pallas as pl
from jax.experimental.pallas import tpu as pltpu
from jax.experimental.pallas import tpu_sc as plsc

B, S, D = 4, 4096, 1024
N = B * S

_NC, _NS = 2, 16
_NW = _NC * _NS
_IDX_COLS = 128
_ROWS = N // _IDX_COLS
_ROWS_PER_W = _ROWS // _NW


_TOK_PER_W = N // _NW
_CHUNKS_PER_W = _TOK_PER_W // _IDX_COLS
_W_PER_B = _NW // B


def _flag_gather_sc(token_ids, is_continuation):
  mesh = plsc.VectorSubcoreMesh(core_axis_name="c", subcore_axis_name="s")

  @functools.partial(
      pl.kernel,
      mesh=mesh,
      out_type=jax.ShapeDtypeStruct((_ROWS, _IDX_COLS), jnp.int32),
      scratch_types=[
          pltpu.VMEM((_TOK_PER_W,), jnp.int32),
          pltpu.VMEM((_CHUNKS_PER_W, _IDX_COLS), jnp.int32),
          pltpu.SemaphoreType.DMA,
      ],
  )
  def k(tok_hbm, cont_hbm, flags_hbm, idx_v, flg_v, sem):
    wid = lax.axis_index("s") * _NC + lax.axis_index("c")
    b = wid // _W_PER_B
    col0 = (wid % _W_PER_B) * _TOK_PER_W
    pltpu.sync_copy(tok_hbm.at[b, pl.ds(col0, _TOK_PER_W)], idx_v)
    copies = []
    for c in range(_CHUNKS_PER_W):
      copies.append(
          pltpu.async_copy(
              cont_hbm.at[idx_v.at[pl.ds(c * _IDX_COLS, _IDX_COLS)]],
              flg_v.at[c], sem))
    for cp in copies:
      cp.wait()
    pltpu.sync_copy(flg_v, flags_hbm.at[pl.ds(wid * _ROWS_PER_W, _ROWS_PER_W)])

  return k(token_ids, is_continuation)


_BS = 2048
_FR = _BS // _IDX_COLS


def _add_kernel(emb_ref, flg_ref, w_ref, out_ref):
  w0 = w_ref[0:1, :]
  w1 = w_ref[1:2, :]
  f = flg_ref[...].astype(jnp.float32)
  eye = (lax.broadcasted_iota(jnp.int32, (_FR, _FR), 0) ==
         lax.broadcasted_iota(jnp.int32, (_FR, _FR), 1)).astype(jnp.float32)
  ft = lax.dot_general(f, eye, (((0,), (0,)), ((), ())),
                       preferred_element_type=jnp.float32)
  for r in range(_FR):
    pred = ft[:, r:r + 1] != 0.0
    rows = pl.ds(r * _IDX_COLS, _IDX_COLS)
    out_ref[rows, :] = emb_ref[rows, :] + jnp.where(pred, w1, w0)


def _flag_add_tc(emb2d, flags, w):
  grid = (N // _BS,)
  return pl.pallas_call(
      _add_kernel,
      grid=grid,
      in_specs=[
          pl.BlockSpec((_BS, D), lambda i: (i, 0)),
          pl.BlockSpec((_FR, _IDX_COLS), lambda i: (i, 0)),
          pl.BlockSpec((2, D), lambda i: (0, 0)),
      ],
      out_specs=pl.BlockSpec((_BS, D), lambda i: (i, 0)),
      out_shape=jax.ShapeDtypeStruct((N, D), jnp.float32),
  )(emb2d, flags, w)


@jax.jit
def kernel(subword_embeds, token_ids, is_continuation, cont_emb_weight):
  flags = _flag_gather_sc(token_ids.astype(jnp.int32),
                          is_continuation.astype(jnp.int32))
  out = _flag_add_tc(
      subword_embeds.reshape(N, D),
      flags,
      cont_emb_weight,
  )
  return out.reshape(B, S, D)


# speedup vs baseline: 1.0139x; 1.0139x over previous
import functools

import jax
import jax.numpy as jnp
from jax import lax
from jax.experimental import pallas as pl
from jax.experimental.pallas import tpu as pltpu
from jax.experimental.pallas import tpu_sc as plsc

B, S, D = 4, 4096, 1024
N = B * S

_NC, _NS = 1, 16
_NW = _NC * _NS
_IDX_COLS = 128
_ROWS = N // _IDX_COLS
_ROWS_PER_W = _ROWS // _NW


_TOK_PER_W = N // _NW
_CHUNKS_PER_W = _TOK_PER_W // _IDX_COLS
_W_PER_B = _NW // B


def _flag_gather_sc(token_ids, is_continuation):
  mesh = plsc.VectorSubcoreMesh(core_axis_name="c", subcore_axis_name="s", num_cores=1)

  @functools.partial(
      pl.kernel,
      mesh=mesh,
      out_type=jax.ShapeDtypeStruct((_ROWS, _IDX_COLS), jnp.int32),
      scratch_types=[
          pltpu.VMEM((_TOK_PER_W,), jnp.int32),
          pltpu.VMEM((_CHUNKS_PER_W, _IDX_COLS), jnp.int32),
          pltpu.SemaphoreType.DMA,
      ],
  )
  def k(tok_hbm, cont_hbm, flags_hbm, idx_v, flg_v, sem):
    wid = lax.axis_index("s") * _NC + lax.axis_index("c")
    b = wid // _W_PER_B
    col0 = (wid % _W_PER_B) * _TOK_PER_W
    pltpu.sync_copy(tok_hbm.at[b, pl.ds(col0, _TOK_PER_W)], idx_v)
    copies = []
    for c in range(_CHUNKS_PER_W):
      copies.append(
          pltpu.async_copy(
              cont_hbm.at[idx_v.at[pl.ds(c * _IDX_COLS, _IDX_COLS)]],
              flg_v.at[c], sem))
    for cp in copies:
      cp.wait()
    pltpu.sync_copy(flg_v, flags_hbm.at[pl.ds(wid * _ROWS_PER_W, _ROWS_PER_W)])

  return k(token_ids, is_continuation)


_BS = 2048
_FR = _BS // _IDX_COLS


def _add_kernel(emb_ref, flg_ref, w_ref, out_ref):
  w0 = w_ref[0:1, :]
  w1 = w_ref[1:2, :]
  f = flg_ref[...].astype(jnp.float32)
  eye = (lax.broadcasted_iota(jnp.int32, (_FR, _FR), 0) ==
         lax.broadcasted_iota(jnp.int32, (_FR, _FR), 1)).astype(jnp.float32)
  ft = lax.dot_general(f, eye, (((0,), (0,)), ((), ())),
                       preferred_element_type=jnp.float32)
  for r in range(_FR):
    pred = ft[:, r:r + 1] != 0.0
    rows = pl.ds(r * _IDX_COLS, _IDX_COLS)
    out_ref[rows, :] = emb_ref[rows, :] + jnp.where(pred, w1, w0)


def _flag_add_tc(emb2d, flags, w):
  grid = (N // _BS,)
  return pl.pallas_call(
      _add_kernel,
      grid=grid,
      in_specs=[
          pl.BlockSpec((_BS, D), lambda i: (i, 0)),
          pl.BlockSpec((_FR, _IDX_COLS), lambda i: (i, 0)),
          pl.BlockSpec((2, D), lambda i: (0, 0)),
      ],
      out_specs=pl.BlockSpec((_BS, D), lambda i: (i, 0)),
      out_shape=jax.ShapeDtypeStruct((N, D), jnp.float32),
  )(emb2d, flags, w)


@jax.jit
def kernel(subword_embeds, token_ids, is_continuation, cont_emb_weight):
  flags = _flag_gather_sc(token_ids.astype(jnp.int32),
                          is_continuation.astype(jnp.int32))
  out = _flag_add_tc(
      subword_embeds.reshape(N, D),
      flags,
      cont_emb_weight,
  )
  return out.reshape(B, S, D)
